# baseline (speedup 1.0000x reference)
import functools

import jax
import jax.numpy as jnp
from jax import lax
from jax.experimental import pallas as pl
from jax.experimental.pallas import tpu as pltpu
from jax.experimental.pallas import tpu_sc as plsc

N = 10000
E = 160000
D = 256
HALF = 128

N_PAD = 10240
E_PAD = 163840
EROWS = E_PAD // 128
ROWS_PER_TILE_DEG = EROWS // 32

SCH = 32
SROWS = E_PAD // SCH
SPT = SROWS // 16
NB = 2
NHALF = 5120
ACC_R = 5128
TRASH = 5120

_mesh = plsc.VectorSubcoreMesh(
    core_axis_name="c", subcore_axis_name="s", num_cores=2, num_subcores=16
)


@functools.partial(
    pl.kernel,
    out_type=jax.ShapeDtypeStruct((2, N_PAD), jnp.float32),
    mesh=_mesh,
    scratch_types=[
        pltpu.VMEM((ROWS_PER_TILE_DEG, 128), jnp.int32),
        pltpu.VMEM((128,), jnp.float32),
        pltpu.VMEM((640,), jnp.float32),
        pltpu.VMEM_SHARED((N_PAD,), jnp.float32),
    ],
)
def _deg_kernel(dst_hbm, out_hbm, idx_v, ones_v, zeros_v, acc):
    c = lax.axis_index("c")
    s = lax.axis_index("s")
    for k in range(8):
        ones_v[pl.ds(k * 16, 16)] = jnp.full((16,), 1.0, jnp.float32)
    for k in range(40):
        zeros_v[pl.ds(k * 16, 16)] = jnp.zeros((16,), jnp.float32)
    pltpu.sync_copy(zeros_v, acc.at[pl.ds(s * 640, 640)])
    pltpu.sync_copy(
        dst_hbm.at[pl.ds((c * 16 + s) * ROWS_PER_TILE_DEG, ROWS_PER_TILE_DEG)], idx_v
    )
    plsc.subcore_barrier()

    def step(j, carry):
        pltpu.sync_copy(ones_v, acc.at[idx_v.at[j]], add=True)
        return carry

    lax.fori_loop(0, ROWS_PER_TILE_DEG, step, 0)
    plsc.subcore_barrier()
    pltpu.sync_copy(acc.at[pl.ds(s * 640, 640)], out_hbm.at[c, pl.ds(s * 640, 640)])


@functools.partial(
    pl.kernel,
    out_type=jax.ShapeDtypeStruct((2, N, HALF), jnp.float32),
    mesh=_mesh,
    scratch_types=[
        pltpu.VMEM((NB, SCH), jnp.int32),
        pltpu.VMEM((NB, SCH), jnp.int32),
        pltpu.VMEM((NB, SCH), jnp.int32),
        pltpu.VMEM((NB, SCH, HALF), jnp.float32),
        pltpu.VMEM_SHARED((N, HALF), jnp.float32),
        pltpu.VMEM_SHARED((ACC_R, HALF), jnp.float32),
        [pltpu.SemaphoreType.DMA] * NB,
        [pltpu.SemaphoreType.DMA] * NB,
        [pltpu.SemaphoreType.DMA] * NB,
    ],
)
def _scatter_kernel(
    h_hbm, src_hbm, dst_hbm, out_hbm,
    si_v, di_v, dj_v, rows_v, hs, acc, gsems, dsems, ssems,
):
    c = lax.axis_index("c")
    s = lax.axis_index("s")

    @pl.when(s < 15)
    def _():
        pltpu.sync_copy(h_hbm.at[c, pl.ds(s * 640, 640)], hs.at[pl.ds(s * 640, 640)])

    @pl.when(s == 15)
    def _():
        pltpu.sync_copy(h_hbm.at[c, pl.ds(9600, 400)], hs.at[pl.ds(9600, 400)])

    base = s * SPT

    def fetch_si(b, j):
        pltpu.async_copy(src_hbm.at[base + j], si_v.at[b], ssems[b])

    def wait_si(b):
        pltpu.make_async_copy(src_hbm.at[0], si_v.at[b], ssems[b]).wait()

    def gather(b, j):
        pltpu.async_copy(hs.at[si_v.at[b]], rows_v.at[b], gsems[b])
        pltpu.async_copy(dst_hbm.at[base + j], di_v.at[b], dsems[b])

    def drain(b):
        pltpu.make_async_copy(hs.at[si_v.at[b]], rows_v.at[b], gsems[b]).wait()
        pltpu.make_async_copy(dst_hbm.at[0], di_v.at[b], dsems[b]).wait()

    for p in range(2):
        lo = p * NHALF

        if p == 0:
            pltpu.sync_copy(
                h_hbm.at[c, pl.ds(s * 320, 320)], acc.at[pl.ds(s * 320, 320)]
            )
        else:

            @pl.when(s < 15)
            def _():
                pltpu.sync_copy(
                    h_hbm.at[c, pl.ds(NHALF + s * 320, 320)],
                    acc.at[pl.ds(s * 320, 320)],
                )

            @pl.when(s == 15)
            def _():
                pltpu.sync_copy(
                    h_hbm.at[c, pl.ds(NHALF + 4800, 80)], acc.at[pl.ds(4800, 80)]
                )

        for b in range(NB):
            fetch_si(b, b)
        plsc.subcore_barrier()
        for b in range(NB - 1):
            wait_si(b)
            gather(b, b)

        def step(k, carry):
            j = NB * k
            for b in range(NB):
                drain(b)
                for v in range(SCH // 16):
                    d2 = di_v[b, pl.ds(v * 16, 16)] - lo
                    ok = (d2 >= 0) & (d2 < NHALF)
                    dj_v[b, pl.ds(v * 16, 16)] = jnp.where(ok, d2, TRASH)
                pltpu.sync_copy(rows_v.at[b], acc.at[dj_v.at[b]], add=True)
                fetch_si(b, lax.rem(j + b + NB, SPT))
                bn = (b + NB - 1) % NB
                wait_si(bn)
                gather(bn, lax.rem(j + b + NB - 1, SPT))
            return carry

        lax.fori_loop(0, SPT // NB, step, 0)
        for b in range(NB - 1):
            drain(b)
        wait_si((SPT + NB - 1) % NB)
        plsc.subcore_barrier()

        if p == 0:
            pltpu.sync_copy(
                acc.at[pl.ds(s * 320, 320)], out_hbm.at[c, pl.ds(s * 320, 320)]
            )
        else:

            @pl.when(s < 15)
            def _():
                pltpu.sync_copy(
                    acc.at[pl.ds(s * 320, 320)],
                    out_hbm.at[c, pl.ds(NHALF + s * 320, 320)],
                )

            @pl.when(s == 15)
            def _():
                pltpu.sync_copy(
                    acc.at[pl.ds(4800, 80)], out_hbm.at[c, pl.ds(NHALF + 4800, 80)]
                )


R = 512
GRID_I = (N + R - 1) // R


def _dinv(deg_ref):
    return lax.rsqrt(1.0 + deg_ref[0, :] + deg_ref[1, :])[:, None]


def _mm0_body(x_ref, w_ref, deg_ref, out_ref):
    h = jnp.dot(x_ref[...], w_ref[...], preferred_element_type=jnp.float32)
    out_ref[0] = h * _dinv(deg_ref)


_mm0 = pl.pallas_call(
    _mm0_body,
    grid=(GRID_I, 2),
    in_specs=[
        pl.BlockSpec((R, D), lambda i, j: (i, 0)),
        pl.BlockSpec((D, HALF), lambda i, j: (0, j)),
        pl.BlockSpec((2, R), lambda i, j: (0, i)),
    ],
    out_specs=pl.BlockSpec((1, R, HALF), lambda i, j: (j, i, 0)),
    out_shape=jax.ShapeDtypeStruct((2, N, HALF), jnp.float32),
)


def _mm1_body(s0_ref, w_ref, b_ref, deg_ref, out_ref):
    dinv = _dinv(deg_ref)
    x1a = s0_ref[0] * dinv + b_ref[0, 0:HALF][None, :]
    x1b = s0_ref[1] * dinv + b_ref[0, HALF:D][None, :]
    h = jnp.dot(x1a, w_ref[0:HALF, :], preferred_element_type=jnp.float32)
    h += jnp.dot(x1b, w_ref[HALF:D, :], preferred_element_type=jnp.float32)
    out_ref[0] = h * dinv


_mm1 = pl.pallas_call(
    _mm1_body,
    grid=(GRID_I, 2),
    in_specs=[
        pl.BlockSpec((2, R, HALF), lambda i, j: (0, i, 0)),
        pl.BlockSpec((D, HALF), lambda i, j: (0, j)),
        pl.BlockSpec((1, D), lambda i, j: (0, 0)),
        pl.BlockSpec((2, R), lambda i, j: (0, i)),
    ],
    out_specs=pl.BlockSpec((1, R, HALF), lambda i, j: (j, i, 0)),
    out_shape=jax.ShapeDtypeStruct((2, N, HALF), jnp.float32),
)


def _fin_body(s1_ref, b_ref, deg_ref, out_ref):
    dinv = _dinv(deg_ref)
    a = s1_ref[0] * dinv + b_ref[0, 0:HALF][None, :]
    b = s1_ref[1] * dinv + b_ref[0, HALF:D][None, :]
    out_ref[...] = jnp.concatenate([a, b], axis=1)


_fin = pl.pallas_call(
    _fin_body,
    grid=(GRID_I,),
    in_specs=[
        pl.BlockSpec((2, R, HALF), lambda i: (0, i, 0)),
        pl.BlockSpec((1, D), lambda i: (0, 0)),
        pl.BlockSpec((2, R), lambda i: (0, i)),
    ],
    out_specs=pl.BlockSpec((R, D), lambda i: (i, 0)),
    out_shape=jax.ShapeDtypeStruct((N, D), jnp.float32),
)


def kernel(node_features, edge_index, W0, b0, W1, b1):
    src = edge_index[0].astype(jnp.int32)
    dst = edge_index[1].astype(jnp.int32)
    pad = E_PAD - E
    src_p = jnp.concatenate([src, jnp.zeros((pad,), jnp.int32)])
    dst_p = jnp.concatenate([dst, jnp.full((pad,), N, jnp.int32)])
    dst2d = dst_p.reshape(EROWS, 128)
    src32 = src_p.reshape(SROWS, SCH)
    dst32 = dst_p.reshape(SROWS, SCH)

    deg = _deg_kernel(dst2d)
    h0 = _mm0(node_features, W0, deg)
    s0 = _scatter_kernel(h0, src32, dst32)
    h1 = _mm1(s0, W1, b0.reshape(1, D), deg)
    s1 = _scatter_kernel(h1, src32, dst32)
    return _fin(s1, b1.reshape(1, D), deg)


# speedup vs baseline: 1.5397x; 1.5397x over previous
import functools

import jax
import jax.numpy as jnp
from jax import lax
from jax.experimental import pallas as pl
from jax.experimental.pallas import tpu as pltpu
from jax.experimental.pallas import tpu_sc as plsc

N = 10000
E = 160000
D = 256
HALF = 128

N_PAD = 10240
E_PAD = 163840
EROWS = E_PAD // 128
ROWS_PER_TILE = EROWS // 16
ROWS_PER_TILE_DEG = EROWS // 32

_mesh = plsc.VectorSubcoreMesh(
    core_axis_name="c", subcore_axis_name="s", num_cores=2, num_subcores=16
)


@functools.partial(
    pl.kernel,
    out_type=jax.ShapeDtypeStruct((2, N_PAD), jnp.float32),
    mesh=_mesh,
    scratch_types=[
        pltpu.VMEM((ROWS_PER_TILE_DEG, 128), jnp.int32),
        pltpu.VMEM((128,), jnp.float32),
        pltpu.VMEM((640,), jnp.float32),
        pltpu.VMEM_SHARED((N_PAD,), jnp.float32),
    ],
)
def _deg_kernel(dst_hbm, out_hbm, idx_v, ones_v, zeros_v, acc):
    c = lax.axis_index("c")
    s = lax.axis_index("s")
    for k in range(8):
        ones_v[pl.ds(k * 16, 16)] = jnp.full((16,), 1.0, jnp.float32)
    for k in range(40):
        zeros_v[pl.ds(k * 16, 16)] = jnp.zeros((16,), jnp.float32)
    pltpu.sync_copy(zeros_v, acc.at[pl.ds(s * 640, 640)])
    pltpu.sync_copy(
        dst_hbm.at[pl.ds((c * 16 + s) * ROWS_PER_TILE_DEG, ROWS_PER_TILE_DEG)], idx_v
    )
    plsc.subcore_barrier()

    def step(j, carry):
        pltpu.sync_copy(ones_v, acc.at[idx_v.at[j]], add=True)
        return carry

    lax.fori_loop(0, ROWS_PER_TILE_DEG, step, 0)
    plsc.subcore_barrier()
    pltpu.sync_copy(acc.at[pl.ds(s * 640, 640)], out_hbm.at[c, pl.ds(s * 640, 640)])


@functools.partial(
    pl.kernel,
    out_type=jax.ShapeDtypeStruct((2, N, HALF), jnp.float32),
    mesh=_mesh,
    scratch_types=[
        pltpu.VMEM((ROWS_PER_TILE, 128), jnp.int32),
        pltpu.VMEM((2, 128), jnp.int32),
        pltpu.VMEM((2, 128, HALF), jnp.float32),
        pltpu.VMEM_SHARED((N_PAD, HALF), jnp.float32),
        pltpu.SemaphoreType.DMA,
        pltpu.SemaphoreType.DMA,
        pltpu.SemaphoreType.DMA,
        pltpu.SemaphoreType.DMA,
    ],
)
def _scatter_kernel(
    h_hbm, src_hbm, dst_hbm, out_hbm, si_v, di_v, rows_v, acc, g0, g1, d0, d1
):
    c = lax.axis_index("c")
    s = lax.axis_index("s")

    @pl.when(s < 15)
    def _():
        pltpu.sync_copy(h_hbm.at[c, pl.ds(s * 640, 640)], acc.at[pl.ds(s * 640, 640)])

    @pl.when(s == 15)
    def _():
        pltpu.sync_copy(h_hbm.at[c, pl.ds(9600, 400)], acc.at[pl.ds(9600, 400)])

    pltpu.sync_copy(src_hbm.at[pl.ds(s * ROWS_PER_TILE, ROWS_PER_TILE)], si_v)
    plsc.subcore_barrier()

    gsems = (g0, g1)
    dsems = (d0, d1)
    base = s * ROWS_PER_TILE

    def fetch(b, j):
        pltpu.async_copy(h_hbm.at[c].at[si_v.at[j]], rows_v.at[b], gsems[b])
        pltpu.async_copy(dst_hbm.at[base + j], di_v.at[b], dsems[b])

    def drain(b):
        pltpu.make_async_copy(h_hbm.at[c].at[si_v.at[0]], rows_v.at[b], gsems[b]).wait()
        pltpu.make_async_copy(dst_hbm.at[0], di_v.at[b], dsems[b]).wait()

    fetch(0, 0)

    def step(k, carry):
        j = 2 * k
        for b in range(2):
            fetch(1 - b, lax.rem(j + b + 1, ROWS_PER_TILE))
            drain(b)
            pltpu.sync_copy(rows_v.at[b], acc.at[di_v.at[b]], add=True)
        return carry

    lax.fori_loop(0, ROWS_PER_TILE // 2, step, 0)
    drain(0)
    plsc.subcore_barrier()

    @pl.when(s < 15)
    def _():
        pltpu.sync_copy(acc.at[pl.ds(s * 640, 640)], out_hbm.at[c, pl.ds(s * 640, 640)])

    @pl.when(s == 15)
    def _():
        pltpu.sync_copy(acc.at[pl.ds(9600, 400)], out_hbm.at[c, pl.ds(9600, 400)])


R = 512
GRID_I = (N + R - 1) // R


def _dinv(deg_ref):
    return lax.rsqrt(1.0 + deg_ref[0, :] + deg_ref[1, :])[:, None]


def _mm0_body(x_ref, w_ref, deg_ref, out_ref):
    h = jnp.dot(x_ref[...], w_ref[...], preferred_element_type=jnp.float32)
    out_ref[0] = h * _dinv(deg_ref)


_mm0 = pl.pallas_call(
    _mm0_body,
    grid=(GRID_I, 2),
    in_specs=[
        pl.BlockSpec((R, D), lambda i, j: (i, 0)),
        pl.BlockSpec((D, HALF), lambda i, j: (0, j)),
        pl.BlockSpec((2, R), lambda i, j: (0, i)),
    ],
    out_specs=pl.BlockSpec((1, R, HALF), lambda i, j: (j, i, 0)),
    out_shape=jax.ShapeDtypeStruct((2, N, HALF), jnp.float32),
)


def _mm1_body(s0_ref, w_ref, b_ref, deg_ref, out_ref):
    dinv = _dinv(deg_ref)
    x1a = s0_ref[0] * dinv + b_ref[0, 0:HALF][None, :]
    x1b = s0_ref[1] * dinv + b_ref[0, HALF:D][None, :]
    h = jnp.dot(x1a, w_ref[0:HALF, :], preferred_element_type=jnp.float32)
    h += jnp.dot(x1b, w_ref[HALF:D, :], preferred_element_type=jnp.float32)
    out_ref[0] = h * dinv


_mm1 = pl.pallas_call(
    _mm1_body,
    grid=(GRID_I, 2),
    in_specs=[
        pl.BlockSpec((2, R, HALF), lambda i, j: (0, i, 0)),
        pl.BlockSpec((D, HALF), lambda i, j: (0, j)),
        pl.BlockSpec((1, D), lambda i, j: (0, 0)),
        pl.BlockSpec((2, R), lambda i, j: (0, i)),
    ],
    out_specs=pl.BlockSpec((1, R, HALF), lambda i, j: (j, i, 0)),
    out_shape=jax.ShapeDtypeStruct((2, N, HALF), jnp.float32),
)


def _fin_body(s1_ref, b_ref, deg_ref, out_ref):
    dinv = _dinv(deg_ref)
    a = s1_ref[0] * dinv + b_ref[0, 0:HALF][None, :]
    b = s1_ref[1] * dinv + b_ref[0, HALF:D][None, :]
    out_ref[...] = jnp.concatenate([a, b], axis=1)


_fin = pl.pallas_call(
    _fin_body,
    grid=(GRID_I,),
    in_specs=[
        pl.BlockSpec((2, R, HALF), lambda i: (0, i, 0)),
        pl.BlockSpec((1, D), lambda i: (0, 0)),
        pl.BlockSpec((2, R), lambda i: (0, i)),
    ],
    out_specs=pl.BlockSpec((R, D), lambda i: (i, 0)),
    out_shape=jax.ShapeDtypeStruct((N, D), jnp.float32),
)


def kernel(node_features, edge_index, W0, b0, W1, b1):
    src = edge_index[0].astype(jnp.int32)
    dst = edge_index[1].astype(jnp.int32)
    pad = E_PAD - E
    src2d = jnp.concatenate([src, jnp.zeros((pad,), jnp.int32)]).reshape(EROWS, 128)
    dst2d = jnp.concatenate([dst, jnp.full((pad,), N, jnp.int32)]).reshape(EROWS, 128)

    deg = _deg_kernel(dst2d)
    h0 = _mm0(node_features, W0, deg)
    s0 = _scatter_kernel(h0, src2d, dst2d)
    h1 = _mm1(s0, W1, b0.reshape(1, D), deg)
    s1 = _scatter_kernel(h1, src2d, dst2d)
    return _fin(s1, b1.reshape(1, D), deg)


# speedup vs baseline: 1.5988x; 1.0384x over previous
import functools

import jax
import jax.numpy as jnp
from jax import lax
from jax.experimental import pallas as pl
from jax.experimental.pallas import tpu as pltpu
from jax.experimental.pallas import tpu_sc as plsc

N = 10000
E = 160000
D = 256
HALF = 128

N_PAD = 10240
E_PAD = 163840
EROWS = E_PAD // 128
ROWS_PER_TILE = EROWS // 16
ROWS_PER_TILE_DEG = EROWS // 32

_mesh = plsc.VectorSubcoreMesh(
    core_axis_name="c", subcore_axis_name="s", num_cores=2, num_subcores=16
)


@functools.partial(
    pl.kernel,
    out_type=jax.ShapeDtypeStruct((2, N_PAD), jnp.float32),
    mesh=_mesh,
    scratch_types=[
        pltpu.VMEM((ROWS_PER_TILE_DEG, 128), jnp.int32),
        pltpu.VMEM((128,), jnp.float32),
        pltpu.VMEM((640,), jnp.float32),
        pltpu.VMEM_SHARED((N_PAD,), jnp.float32),
    ],
)
def _deg_kernel(dst_hbm, out_hbm, idx_v, ones_v, zeros_v, acc):
    c = lax.axis_index("c")
    s = lax.axis_index("s")
    for k in range(8):
        ones_v[pl.ds(k * 16, 16)] = jnp.full((16,), 1.0, jnp.float32)
    for k in range(40):
        zeros_v[pl.ds(k * 16, 16)] = jnp.zeros((16,), jnp.float32)
    pltpu.sync_copy(zeros_v, acc.at[pl.ds(s * 640, 640)])
    pltpu.sync_copy(
        dst_hbm.at[pl.ds((c * 16 + s) * ROWS_PER_TILE_DEG, ROWS_PER_TILE_DEG)], idx_v
    )
    plsc.subcore_barrier()

    def step(j, carry):
        pltpu.sync_copy(ones_v, acc.at[idx_v.at[j]], add=True)
        return carry

    lax.fori_loop(0, ROWS_PER_TILE_DEG, step, 0)
    plsc.subcore_barrier()
    pltpu.sync_copy(acc.at[pl.ds(s * 640, 640)], out_hbm.at[c, pl.ds(s * 640, 640)])


@functools.partial(
    pl.kernel,
    out_type=jax.ShapeDtypeStruct((2, N, HALF), jnp.float32),
    mesh=_mesh,
    scratch_types=[
        pltpu.VMEM((ROWS_PER_TILE, 128), jnp.int32),
        pltpu.VMEM((2, 128), jnp.int32),
        pltpu.VMEM((2, 128, HALF), jnp.float32),
        pltpu.VMEM_SHARED((N_PAD, HALF), jnp.float32),
        pltpu.SemaphoreType.DMA,
        pltpu.SemaphoreType.DMA,
        pltpu.SemaphoreType.DMA,
        pltpu.SemaphoreType.DMA,
    ],
)
def _scatter_kernel(
    h_hbm, src_hbm, dst_hbm, out_hbm, si_v, di_v, rows_v, acc, g0, g1, d0, d1
):
    c = lax.axis_index("c")
    s = lax.axis_index("s")

    @pl.when(s < 15)
    def _():
        pltpu.sync_copy(h_hbm.at[c, pl.ds(s * 640, 640)], acc.at[pl.ds(s * 640, 640)])

    @pl.when(s == 15)
    def _():
        pltpu.sync_copy(h_hbm.at[c, pl.ds(9600, 400)], acc.at[pl.ds(9600, 400)])

    pltpu.sync_copy(src_hbm.at[pl.ds(s * ROWS_PER_TILE, ROWS_PER_TILE)], si_v)
    plsc.subcore_barrier()

    gsems = (g0, g1)
    dsems = (d0, d1)
    base = s * ROWS_PER_TILE

    def fetch(b, j):
        pltpu.async_copy(h_hbm.at[c].at[si_v.at[j]], rows_v.at[b], gsems[b])
        pltpu.async_copy(dst_hbm.at[base + j], di_v.at[b], dsems[b])

    def drain(b):
        pltpu.make_async_copy(h_hbm.at[c].at[si_v.at[0]], rows_v.at[b], gsems[b]).wait()
        pltpu.make_async_copy(dst_hbm.at[0], di_v.at[b], dsems[b]).wait()

    fetch(0, 0)

    def step(k, carry):
        j = 2 * k
        for b in range(2):
            fetch(1 - b, lax.rem(j + b + 1, ROWS_PER_TILE))
            drain(b)
            pltpu.sync_copy(rows_v.at[b], acc.at[di_v.at[b]], add=True)
        return carry

    lax.fori_loop(0, ROWS_PER_TILE // 2, step, 0)
    drain(0)
    plsc.subcore_barrier()

    @pl.when(s < 15)
    def _():
        pltpu.sync_copy(acc.at[pl.ds(s * 640, 640)], out_hbm.at[c, pl.ds(s * 640, 640)])

    @pl.when(s == 15)
    def _():
        pltpu.sync_copy(acc.at[pl.ds(9600, 400)], out_hbm.at[c, pl.ds(9600, 400)])


R = 512
GRID_I = (N + R - 1) // R


def _dinv(deg_ref):
    return lax.rsqrt(1.0 + deg_ref[0, :] + deg_ref[1, :])[:, None]


def _mm0_body(x_ref, w_ref, deg_ref, out_ref):
    h = jnp.dot(x_ref[...], w_ref[...], preferred_element_type=jnp.float32)
    dinv = _dinv(deg_ref)
    out_ref[0] = h[:, 0:HALF] * dinv
    out_ref[1] = h[:, HALF:D] * dinv


_mm0 = pl.pallas_call(
    _mm0_body,
    grid=(GRID_I,),
    in_specs=[
        pl.BlockSpec((R, D), lambda i: (i, 0)),
        pl.BlockSpec((D, D), lambda i: (0, 0)),
        pl.BlockSpec((2, R), lambda i: (0, i)),
    ],
    out_specs=pl.BlockSpec((2, R, HALF), lambda i: (0, i, 0)),
    out_shape=jax.ShapeDtypeStruct((2, N, HALF), jnp.float32),
)


def _mm1_body(s0_ref, w_ref, b_ref, deg_ref, out_ref):
    dinv = _dinv(deg_ref)
    x1a = s0_ref[0] * dinv + b_ref[0, 0:HALF][None, :]
    x1b = s0_ref[1] * dinv + b_ref[0, HALF:D][None, :]
    h = jnp.dot(x1a, w_ref[0:HALF, :], preferred_element_type=jnp.float32)
    h += jnp.dot(x1b, w_ref[HALF:D, :], preferred_element_type=jnp.float32)
    out_ref[0] = h[:, 0:HALF] * dinv
    out_ref[1] = h[:, HALF:D] * dinv


_mm1 = pl.pallas_call(
    _mm1_body,
    grid=(GRID_I,),
    in_specs=[
        pl.BlockSpec((2, R, HALF), lambda i: (0, i, 0)),
        pl.BlockSpec((D, D), lambda i: (0, 0)),
        pl.BlockSpec((1, D), lambda i: (0, 0)),
        pl.BlockSpec((2, R), lambda i: (0, i)),
    ],
    out_specs=pl.BlockSpec((2, R, HALF), lambda i: (0, i, 0)),
    out_shape=jax.ShapeDtypeStruct((2, N, HALF), jnp.float32),
)


def _fin_body(s1_ref, b_ref, deg_ref, out_ref):
    dinv = _dinv(deg_ref)
    a = s1_ref[0] * dinv + b_ref[0, 0:HALF][None, :]
    b = s1_ref[1] * dinv + b_ref[0, HALF:D][None, :]
    out_ref[...] = jnp.concatenate([a, b], axis=1)


_fin = pl.pallas_call(
    _fin_body,
    grid=(GRID_I,),
    in_specs=[
        pl.BlockSpec((2, R, HALF), lambda i: (0, i, 0)),
        pl.BlockSpec((1, D), lambda i: (0, 0)),
        pl.BlockSpec((2, R), lambda i: (0, i)),
    ],
    out_specs=pl.BlockSpec((R, D), lambda i: (i, 0)),
    out_shape=jax.ShapeDtypeStruct((N, D), jnp.float32),
)


def kernel(node_features, edge_index, W0, b0, W1, b1):
    src = edge_index[0].astype(jnp.int32)
    dst = edge_index[1].astype(jnp.int32)
    pad = E_PAD - E
    src2d = jnp.concatenate([src, jnp.zeros((pad,), jnp.int32)]).reshape(EROWS, 128)
    dst2d = jnp.concatenate([dst, jnp.full((pad,), N, jnp.int32)]).reshape(EROWS, 128)

    deg = _deg_kernel(dst2d)
    h0 = _mm0(node_features, W0, deg)
    s0 = _scatter_kernel(h0, src2d, dst2d)
    h1 = _mm1(s0, W1, b0.reshape(1, D), deg)
    s1 = _scatter_kernel(h1, src2d, dst2d)
    return _fin(s1, b1.reshape(1, D), deg)


# speedup vs baseline: 1.6554x; 1.0354x over previous
import functools

import jax
import jax.numpy as jnp
from jax import lax
from jax.experimental import pallas as pl
from jax.experimental.pallas import tpu as pltpu
from jax.experimental.pallas import tpu_sc as plsc

N = 10000
E = 160000
D = 256
HALF = 128

N_PAD = 10240
E_PAD = 163840
EROWS = E_PAD // 128
ROWS_PER_TILE = EROWS // 16
ROWS_PER_TILE_DEG = EROWS // 32

_mesh = plsc.VectorSubcoreMesh(
    core_axis_name="c", subcore_axis_name="s", num_cores=2, num_subcores=16
)


@functools.partial(
    pl.kernel,
    out_type=jax.ShapeDtypeStruct((2, N_PAD), jnp.float32),
    mesh=_mesh,
    scratch_types=[
        pltpu.VMEM((ROWS_PER_TILE_DEG, 128), jnp.int32),
        pltpu.VMEM((128,), jnp.float32),
        pltpu.VMEM((640,), jnp.float32),
        pltpu.VMEM_SHARED((N_PAD,), jnp.float32),
    ],
)
def _deg_kernel(dst_hbm, out_hbm, idx_v, ones_v, zeros_v, acc):
    c = lax.axis_index("c")
    s = lax.axis_index("s")
    for k in range(8):
        ones_v[pl.ds(k * 16, 16)] = jnp.full((16,), 1.0, jnp.float32)
    for k in range(40):
        zeros_v[pl.ds(k * 16, 16)] = jnp.zeros((16,), jnp.float32)
    pltpu.sync_copy(zeros_v, acc.at[pl.ds(s * 640, 640)])
    pltpu.sync_copy(
        dst_hbm.at[pl.ds((c * 16 + s) * ROWS_PER_TILE_DEG, ROWS_PER_TILE_DEG)], idx_v
    )
    plsc.subcore_barrier()

    def step(j, carry):
        pltpu.sync_copy(ones_v, acc.at[idx_v.at[j]], add=True)
        return carry

    lax.fori_loop(0, ROWS_PER_TILE_DEG, step, 0)
    plsc.subcore_barrier()
    pltpu.sync_copy(acc.at[pl.ds(s * 640, 640)], out_hbm.at[c, pl.ds(s * 640, 640)])


@functools.partial(
    pl.kernel,
    out_type=jax.ShapeDtypeStruct((2, N, HALF), jnp.float32),
    mesh=_mesh,
    scratch_types=[
        pltpu.VMEM((ROWS_PER_TILE, 128), jnp.int32),
        pltpu.VMEM((2, 128), jnp.int32),
        pltpu.VMEM((2, 128, HALF), jnp.float32),
        pltpu.VMEM_SHARED((N_PAD, HALF), jnp.float32),
        pltpu.SemaphoreType.DMA,
        pltpu.SemaphoreType.DMA,
        pltpu.SemaphoreType.DMA,
        pltpu.SemaphoreType.DMA,
    ],
)
def _scatter_kernel(
    h_hbm, src_hbm, dst_hbm, out_hbm, si_v, di_v, rows_v, acc, g0, g1, d0, d1
):
    c = lax.axis_index("c")
    s = lax.axis_index("s")

    @pl.when(s < 15)
    def _():
        pltpu.sync_copy(h_hbm.at[c, pl.ds(s * 640, 640)], acc.at[pl.ds(s * 640, 640)])

    @pl.when(s == 15)
    def _():
        pltpu.sync_copy(h_hbm.at[c, pl.ds(9600, 400)], acc.at[pl.ds(9600, 400)])

    pltpu.sync_copy(src_hbm.at[pl.ds(s * ROWS_PER_TILE, ROWS_PER_TILE)], si_v)
    plsc.subcore_barrier()

    gsems = (g0, g1)
    dsems = (d0, d1)
    base = s * ROWS_PER_TILE

    def fetch(b, j):
        pltpu.async_copy(h_hbm.at[c].at[si_v.at[j]], rows_v.at[b], gsems[b])
        pltpu.async_copy(dst_hbm.at[base + j], di_v.at[b], dsems[b])

    def drain(b):
        pltpu.make_async_copy(h_hbm.at[c].at[si_v.at[0]], rows_v.at[b], gsems[b]).wait()
        pltpu.make_async_copy(dst_hbm.at[0], di_v.at[b], dsems[b]).wait()

    fetch(0, 0)

    def step(k, carry):
        j = 2 * k
        for b in range(2):
            fetch(1 - b, lax.rem(j + b + 1, ROWS_PER_TILE))
            drain(b)
            pltpu.sync_copy(rows_v.at[b], acc.at[di_v.at[b]], add=True)
        return carry

    lax.fori_loop(0, ROWS_PER_TILE // 2, step, 0)
    drain(0)
    plsc.subcore_barrier()

    @pl.when(s < 15)
    def _():
        pltpu.sync_copy(acc.at[pl.ds(s * 640, 640)], out_hbm.at[c, pl.ds(s * 640, 640)])

    @pl.when(s == 15)
    def _():
        pltpu.sync_copy(acc.at[pl.ds(9600, 400)], out_hbm.at[c, pl.ds(9600, 400)])


R = 1024
GRID_I = (N + R - 1) // R


def _dinv(deg_ref):
    return lax.rsqrt(1.0 + deg_ref[0, :] + deg_ref[1, :])[:, None]


def _mm0_body(x_ref, w_ref, deg_ref, out_ref):
    h = jnp.dot(x_ref[...], w_ref[...], preferred_element_type=jnp.float32)
    dinv = _dinv(deg_ref)
    out_ref[0] = h[:, 0:HALF] * dinv
    out_ref[1] = h[:, HALF:D] * dinv


_mm0 = pl.pallas_call(
    _mm0_body,
    grid=(GRID_I,),
    in_specs=[
        pl.BlockSpec((R, D), lambda i: (i, 0)),
        pl.BlockSpec((D, D), lambda i: (0, 0)),
        pl.BlockSpec((2, R), lambda i: (0, i)),
    ],
    out_specs=pl.BlockSpec((2, R, HALF), lambda i: (0, i, 0)),
    out_shape=jax.ShapeDtypeStruct((2, N, HALF), jnp.float32),
)


def _mm1_body(s0_ref, w_ref, b_ref, deg_ref, out_ref):
    dinv = _dinv(deg_ref)
    x1a = s0_ref[0] * dinv + b_ref[0, 0:HALF][None, :]
    x1b = s0_ref[1] * dinv + b_ref[0, HALF:D][None, :]
    h = jnp.dot(x1a, w_ref[0:HALF, :], preferred_element_type=jnp.float32)
    h += jnp.dot(x1b, w_ref[HALF:D, :], preferred_element_type=jnp.float32)
    out_ref[0] = h[:, 0:HALF] * dinv
    out_ref[1] = h[:, HALF:D] * dinv


_mm1 = pl.pallas_call(
    _mm1_body,
    grid=(GRID_I,),
    in_specs=[
        pl.BlockSpec((2, R, HALF), lambda i: (0, i, 0)),
        pl.BlockSpec((D, D), lambda i: (0, 0)),
        pl.BlockSpec((1, D), lambda i: (0, 0)),
        pl.BlockSpec((2, R), lambda i: (0, i)),
    ],
    out_specs=pl.BlockSpec((2, R, HALF), lambda i: (0, i, 0)),
    out_shape=jax.ShapeDtypeStruct((2, N, HALF), jnp.float32),
)


def _fin_body(s1_ref, b_ref, deg_ref, out_ref):
    dinv = _dinv(deg_ref)
    a = s1_ref[0] * dinv + b_ref[0, 0:HALF][None, :]
    b = s1_ref[1] * dinv + b_ref[0, HALF:D][None, :]
    out_ref[...] = jnp.concatenate([a, b], axis=1)


_fin = pl.pallas_call(
    _fin_body,
    grid=(GRID_I,),
    in_specs=[
        pl.BlockSpec((2, R, HALF), lambda i: (0, i, 0)),
        pl.BlockSpec((1, D), lambda i: (0, 0)),
        pl.BlockSpec((2, R), lambda i: (0, i)),
    ],
    out_specs=pl.BlockSpec((R, D), lambda i: (i, 0)),
    out_shape=jax.ShapeDtypeStruct((N, D), jnp.float32),
)


def kernel(node_features, edge_index, W0, b0, W1, b1):
    src = edge_index[0].astype(jnp.int32)
    dst = edge_index[1].astype(jnp.int32)
    pad = E_PAD - E
    src2d = jnp.concatenate([src, jnp.zeros((pad,), jnp.int32)]).reshape(EROWS, 128)
    dst2d = jnp.concatenate([dst, jnp.full((pad,), N, jnp.int32)]).reshape(EROWS, 128)

    deg = _deg_kernel(dst2d)
    h0 = _mm0(node_features, W0, deg)
    s0 = _scatter_kernel(h0, src2d, dst2d)
    h1 = _mm1(s0, W1, b0.reshape(1, D), deg)
    s1 = _scatter_kernel(h1, src2d, dst2d)
    return _fin(s1, b1.reshape(1, D), deg)


# speedup vs baseline: 1.6676x; 1.0074x over previous
import functools

import jax
import jax.numpy as jnp
from jax import lax
from jax.experimental import pallas as pl
from jax.experimental.pallas import tpu as pltpu
from jax.experimental.pallas import tpu_sc as plsc

N = 10000
E = 160000
D = 256
HALF = 128

N_PAD = 10240
E_PAD = 163840
EROWS = E_PAD // 128
ROWS_PER_TILE = EROWS // 16
ROWS_PER_TILE_DEG = EROWS // 32

_mesh = plsc.VectorSubcoreMesh(
    core_axis_name="c", subcore_axis_name="s", num_cores=2, num_subcores=16
)


@functools.partial(
    pl.kernel,
    out_type=jax.ShapeDtypeStruct((2, N_PAD), jnp.float32),
    mesh=_mesh,
    scratch_types=[
        pltpu.VMEM((ROWS_PER_TILE_DEG, 128), jnp.int32),
        pltpu.VMEM((128,), jnp.float32),
        pltpu.VMEM((640,), jnp.float32),
        pltpu.VMEM_SHARED((N_PAD,), jnp.float32),
    ],
)
def _deg_kernel(dst_hbm, out_hbm, idx_v, ones_v, zeros_v, acc):
    c = lax.axis_index("c")
    s = lax.axis_index("s")
    for k in range(8):
        ones_v[pl.ds(k * 16, 16)] = jnp.full((16,), 1.0, jnp.float32)
    for k in range(40):
        zeros_v[pl.ds(k * 16, 16)] = jnp.zeros((16,), jnp.float32)
    pltpu.sync_copy(zeros_v, acc.at[pl.ds(s * 640, 640)])
    pltpu.sync_copy(
        dst_hbm.at[pl.ds((c * 16 + s) * ROWS_PER_TILE_DEG, ROWS_PER_TILE_DEG)], idx_v
    )
    plsc.subcore_barrier()

    def step(j, carry):
        pltpu.sync_copy(ones_v, acc.at[idx_v.at[j]], add=True)
        return carry

    lax.fori_loop(0, ROWS_PER_TILE_DEG, step, 0)
    plsc.subcore_barrier()
    pltpu.sync_copy(acc.at[pl.ds(s * 640, 640)], out_hbm.at[c, pl.ds(s * 640, 640)])


@functools.partial(
    pl.kernel,
    out_type=jax.ShapeDtypeStruct((2, N, HALF), jnp.float32),
    mesh=_mesh,
    scratch_types=[
        pltpu.VMEM((ROWS_PER_TILE, 128), jnp.int32),
        pltpu.VMEM((2, 128), jnp.int32),
        pltpu.VMEM((2, 128, HALF), jnp.float32),
        pltpu.VMEM_SHARED((N_PAD, HALF), jnp.float32),
        pltpu.SemaphoreType.DMA,
        pltpu.SemaphoreType.DMA,
        pltpu.SemaphoreType.DMA,
        pltpu.SemaphoreType.DMA,
    ],
)
def _scatter_kernel(
    h_hbm, src_hbm, dst_hbm, out_hbm, si_v, di_v, rows_v, acc, g0, g1, d0, d1
):
    c = lax.axis_index("c")
    s = lax.axis_index("s")

    @pl.when(s < 15)
    def _():
        pltpu.sync_copy(h_hbm.at[c, pl.ds(s * 640, 640)], acc.at[pl.ds(s * 640, 640)])

    @pl.when(s == 15)
    def _():
        pltpu.sync_copy(h_hbm.at[c, pl.ds(9600, 400)], acc.at[pl.ds(9600, 400)])

    pltpu.sync_copy(src_hbm.at[pl.ds(s * ROWS_PER_TILE, ROWS_PER_TILE)], si_v)
    plsc.subcore_barrier()

    gsems = (g0, g1)
    dsems = (d0, d1)
    base = s * ROWS_PER_TILE

    def fetch(b, j):
        pltpu.async_copy(h_hbm.at[c].at[si_v.at[j]], rows_v.at[b], gsems[b])
        pltpu.async_copy(dst_hbm.at[base + j], di_v.at[b], dsems[b])

    def drain(b):
        pltpu.make_async_copy(h_hbm.at[c].at[si_v.at[0]], rows_v.at[b], gsems[b]).wait()
        pltpu.make_async_copy(dst_hbm.at[0], di_v.at[b], dsems[b]).wait()

    fetch(0, 0)

    def step(k, carry):
        j = 2 * k
        for b in range(2):
            fetch(1 - b, lax.rem(j + b + 1, ROWS_PER_TILE))
            drain(b)
            pltpu.sync_copy(rows_v.at[b], acc.at[di_v.at[b]], add=True)
        return carry

    lax.fori_loop(0, ROWS_PER_TILE // 2, step, 0)
    drain(0)
    plsc.subcore_barrier()

    @pl.when(s < 15)
    def _():
        pltpu.sync_copy(acc.at[pl.ds(s * 640, 640)], out_hbm.at[c, pl.ds(s * 640, 640)])

    @pl.when(s == 15)
    def _():
        pltpu.sync_copy(acc.at[pl.ds(9600, 400)], out_hbm.at[c, pl.ds(9600, 400)])


R = 2560
GRID_I = (N + R - 1) // R


def _dinv(deg_ref):
    return lax.rsqrt(1.0 + deg_ref[0, :] + deg_ref[1, :])[:, None]


def _mm0_body(x_ref, w_ref, deg_ref, out_ref):
    h = jnp.dot(x_ref[...], w_ref[...], preferred_element_type=jnp.float32)
    dinv = _dinv(deg_ref)
    out_ref[0] = h[:, 0:HALF] * dinv
    out_ref[1] = h[:, HALF:D] * dinv


_mm0 = pl.pallas_call(
    _mm0_body,
    grid=(GRID_I,),
    in_specs=[
        pl.BlockSpec((R, D), lambda i: (i, 0)),
        pl.BlockSpec((D, D), lambda i: (0, 0)),
        pl.BlockSpec((2, R), lambda i: (0, i)),
    ],
    out_specs=pl.BlockSpec((2, R, HALF), lambda i: (0, i, 0)),
    out_shape=jax.ShapeDtypeStruct((2, N, HALF), jnp.float32),
)


def _mm1_body(s0_ref, w_ref, b_ref, deg_ref, out_ref):
    dinv = _dinv(deg_ref)
    x1a = s0_ref[0] * dinv + b_ref[0, 0:HALF][None, :]
    x1b = s0_ref[1] * dinv + b_ref[0, HALF:D][None, :]
    h = jnp.dot(x1a, w_ref[0:HALF, :], preferred_element_type=jnp.float32)
    h += jnp.dot(x1b, w_ref[HALF:D, :], preferred_element_type=jnp.float32)
    out_ref[0] = h[:, 0:HALF] * dinv
    out_ref[1] = h[:, HALF:D] * dinv


_mm1 = pl.pallas_call(
    _mm1_body,
    grid=(GRID_I,),
    in_specs=[
        pl.BlockSpec((2, R, HALF), lambda i: (0, i, 0)),
        pl.BlockSpec((D, D), lambda i: (0, 0)),
        pl.BlockSpec((1, D), lambda i: (0, 0)),
        pl.BlockSpec((2, R), lambda i: (0, i)),
    ],
    out_specs=pl.BlockSpec((2, R, HALF), lambda i: (0, i, 0)),
    out_shape=jax.ShapeDtypeStruct((2, N, HALF), jnp.float32),
)


def _fin_body(s1_ref, b_ref, deg_ref, out_ref):
    dinv = _dinv(deg_ref)
    a = s1_ref[0] * dinv + b_ref[0, 0:HALF][None, :]
    b = s1_ref[1] * dinv + b_ref[0, HALF:D][None, :]
    out_ref[...] = jnp.concatenate([a, b], axis=1)


_fin = pl.pallas_call(
    _fin_body,
    grid=(GRID_I,),
    in_specs=[
        pl.BlockSpec((2, R, HALF), lambda i: (0, i, 0)),
        pl.BlockSpec((1, D), lambda i: (0, 0)),
        pl.BlockSpec((2, R), lambda i: (0, i)),
    ],
    out_specs=pl.BlockSpec((R, D), lambda i: (i, 0)),
    out_shape=jax.ShapeDtypeStruct((N, D), jnp.float32),
)


def kernel(node_features, edge_index, W0, b0, W1, b1):
    src = edge_index[0].astype(jnp.int32)
    dst = edge_index[1].astype(jnp.int32)
    pad = E_PAD - E
    src2d = jnp.concatenate([src, jnp.zeros((pad,), jnp.int32)]).reshape(EROWS, 128)
    dst2d = jnp.concatenate([dst, jnp.full((pad,), N, jnp.int32)]).reshape(EROWS, 128)

    deg = _deg_kernel(dst2d)
    h0 = _mm0(node_features, W0, deg)
    s0 = _scatter_kernel(h0, src2d, dst2d)
    h1 = _mm1(s0, W1, b0.reshape(1, D), deg)
    s1 = _scatter_kernel(h1, src2d, dst2d)
    return _fin(s1, b1.reshape(1, D), deg)


# speedup vs baseline: 1.6760x; 1.0050x over previous
import functools

import jax
import jax.numpy as jnp
from jax import lax
from jax.experimental import pallas as pl
from jax.experimental.pallas import tpu as pltpu
from jax.experimental.pallas import tpu_sc as plsc

N = 10000
E = 160000
D = 256
HALF = 128

N_PAD = 10240
E_PAD = 163840
EROWS = E_PAD // 128
ROWS_PER_TILE = EROWS // 16
ROWS_PER_TILE_DEG = EROWS // 32

_mesh = plsc.VectorSubcoreMesh(
    core_axis_name="c", subcore_axis_name="s", num_cores=2, num_subcores=16
)


@functools.partial(
    pl.kernel,
    out_type=jax.ShapeDtypeStruct((2, N_PAD), jnp.float32),
    mesh=_mesh,
    scratch_types=[
        pltpu.VMEM((ROWS_PER_TILE_DEG, 128), jnp.int32),
        pltpu.VMEM((128,), jnp.float32),
        pltpu.VMEM((640,), jnp.float32),
        pltpu.VMEM_SHARED((N_PAD,), jnp.float32),
    ],
)
def _deg_kernel(dst_hbm, out_hbm, idx_v, ones_v, zeros_v, acc):
    c = lax.axis_index("c")
    s = lax.axis_index("s")
    for k in range(8):
        ones_v[pl.ds(k * 16, 16)] = jnp.full((16,), 1.0, jnp.float32)
    for k in range(40):
        zeros_v[pl.ds(k * 16, 16)] = jnp.zeros((16,), jnp.float32)
    pltpu.sync_copy(zeros_v, acc.at[pl.ds(s * 640, 640)])
    pltpu.sync_copy(
        dst_hbm.at[pl.ds((c * 16 + s) * ROWS_PER_TILE_DEG, ROWS_PER_TILE_DEG)], idx_v
    )
    plsc.subcore_barrier()

    def step(j, carry):
        pltpu.sync_copy(ones_v, acc.at[idx_v.at[j]], add=True)
        return carry

    lax.fori_loop(0, ROWS_PER_TILE_DEG, step, 0)
    plsc.subcore_barrier()
    pltpu.sync_copy(acc.at[pl.ds(s * 640, 640)], out_hbm.at[c, pl.ds(s * 640, 640)])


@functools.partial(
    pl.kernel,
    out_type=jax.ShapeDtypeStruct((2, N, HALF), jnp.float32),
    mesh=_mesh,
    scratch_types=[
        pltpu.VMEM((ROWS_PER_TILE, 128), jnp.int32),
        pltpu.VMEM((2, 128), jnp.int32),
        pltpu.VMEM((2, 128, HALF), jnp.float32),
        pltpu.VMEM_SHARED((N_PAD, HALF), jnp.float32),
        pltpu.SemaphoreType.DMA,
        pltpu.SemaphoreType.DMA,
        pltpu.SemaphoreType.DMA,
        pltpu.SemaphoreType.DMA,
    ],
)
def _scatter_kernel(
    h_hbm, src_hbm, dst_hbm, out_hbm, si_v, di_v, rows_v, acc, g0, g1, d0, d1
):
    c = lax.axis_index("c")
    s = lax.axis_index("s")

    @pl.when(s < 15)
    def _():
        pltpu.sync_copy(h_hbm.at[c, pl.ds(s * 640, 640)], acc.at[pl.ds(s * 640, 640)])

    @pl.when(s == 15)
    def _():
        pltpu.sync_copy(h_hbm.at[c, pl.ds(9600, 400)], acc.at[pl.ds(9600, 400)])

    pltpu.sync_copy(src_hbm.at[pl.ds(s * ROWS_PER_TILE, ROWS_PER_TILE)], si_v)
    plsc.subcore_barrier()

    gsems = (g0, g1)
    dsems = (d0, d1)
    base = s * ROWS_PER_TILE

    def fetch(b, j):
        pltpu.async_copy(h_hbm.at[c].at[si_v.at[j]], rows_v.at[b], gsems[b])
        pltpu.async_copy(dst_hbm.at[base + j], di_v.at[b], dsems[b])

    def drain(b):
        pltpu.make_async_copy(h_hbm.at[c].at[si_v.at[0]], rows_v.at[b], gsems[b]).wait()
        pltpu.make_async_copy(dst_hbm.at[0], di_v.at[b], dsems[b]).wait()

    fetch(0, 0)

    def step(k, carry):
        j = 2 * k
        for b in range(2):
            fetch(1 - b, lax.rem(j + b + 1, ROWS_PER_TILE))
            drain(b)
            pltpu.sync_copy(rows_v.at[b], acc.at[di_v.at[b]], add=True)
        return carry

    lax.fori_loop(0, ROWS_PER_TILE // 2, step, 0)
    drain(0)
    plsc.subcore_barrier()

    @pl.when(s < 15)
    def _():
        pltpu.sync_copy(acc.at[pl.ds(s * 640, 640)], out_hbm.at[c, pl.ds(s * 640, 640)])

    @pl.when(s == 15)
    def _():
        pltpu.sync_copy(acc.at[pl.ds(9600, 400)], out_hbm.at[c, pl.ds(9600, 400)])


R = 10240
GRID_I = (N + R - 1) // R


def _dinv(deg_ref):
    return lax.rsqrt(1.0 + deg_ref[0, :] + deg_ref[1, :])[:, None]


def _mm0_body(x_ref, w_ref, deg_ref, out_ref):
    h = jnp.dot(x_ref[...], w_ref[...], preferred_element_type=jnp.float32)
    dinv = _dinv(deg_ref)
    out_ref[0] = h[:, 0:HALF] * dinv
    out_ref[1] = h[:, HALF:D] * dinv


_mm0 = pl.pallas_call(
    _mm0_body,
    grid=(GRID_I,),
    in_specs=[
        pl.BlockSpec((R, D), lambda i: (i, 0)),
        pl.BlockSpec((D, D), lambda i: (0, 0)),
        pl.BlockSpec((2, R), lambda i: (0, i)),
    ],
    out_specs=pl.BlockSpec((2, R, HALF), lambda i: (0, i, 0)),
    out_shape=jax.ShapeDtypeStruct((2, N, HALF), jnp.float32),
)


def _mm1_body(s0_ref, w_ref, b_ref, deg_ref, out_ref):
    dinv = _dinv(deg_ref)
    x1a = s0_ref[0] * dinv + b_ref[0, 0:HALF][None, :]
    x1b = s0_ref[1] * dinv + b_ref[0, HALF:D][None, :]
    h = jnp.dot(x1a, w_ref[0:HALF, :], preferred_element_type=jnp.float32)
    h += jnp.dot(x1b, w_ref[HALF:D, :], preferred_element_type=jnp.float32)
    out_ref[0] = h[:, 0:HALF] * dinv
    out_ref[1] = h[:, HALF:D] * dinv


_mm1 = pl.pallas_call(
    _mm1_body,
    grid=(GRID_I,),
    in_specs=[
        pl.BlockSpec((2, R, HALF), lambda i: (0, i, 0)),
        pl.BlockSpec((D, D), lambda i: (0, 0)),
        pl.BlockSpec((1, D), lambda i: (0, 0)),
        pl.BlockSpec((2, R), lambda i: (0, i)),
    ],
    out_specs=pl.BlockSpec((2, R, HALF), lambda i: (0, i, 0)),
    out_shape=jax.ShapeDtypeStruct((2, N, HALF), jnp.float32),
)


def _fin_body(s1_ref, b_ref, deg_ref, out_ref):
    dinv = _dinv(deg_ref)
    a = s1_ref[0] * dinv + b_ref[0, 0:HALF][None, :]
    b = s1_ref[1] * dinv + b_ref[0, HALF:D][None, :]
    out_ref[...] = jnp.concatenate([a, b], axis=1)


_fin = pl.pallas_call(
    _fin_body,
    grid=(GRID_I,),
    in_specs=[
        pl.BlockSpec((2, R, HALF), lambda i: (0, i, 0)),
        pl.BlockSpec((1, D), lambda i: (0, 0)),
        pl.BlockSpec((2, R), lambda i: (0, i)),
    ],
    out_specs=pl.BlockSpec((R, D), lambda i: (i, 0)),
    out_shape=jax.ShapeDtypeStruct((N, D), jnp.float32),
)


def kernel(node_features, edge_index, W0, b0, W1, b1):
    src = edge_index[0].astype(jnp.int32)
    dst = edge_index[1].astype(jnp.int32)
    pad = E_PAD - E
    src2d = jnp.concatenate([src, jnp.zeros((pad,), jnp.int32)]).reshape(EROWS, 128)
    dst2d = jnp.concatenate([dst, jnp.full((pad,), N, jnp.int32)]).reshape(EROWS, 128)

    deg = _deg_kernel(dst2d)
    h0 = _mm0(node_features, W0, deg)
    s0 = _scatter_kernel(h0, src2d, dst2d)
    h1 = _mm1(s0, W1, b0.reshape(1, D), deg)
    s1 = _scatter_kernel(h1, src2d, dst2d)
    return _fin(s1, b1.reshape(1, D), deg)
